# Initial kernel scaffold; baseline (speedup 1.0000x reference)
import jax
import jax.numpy as jnp
from jax.experimental import pallas as pl


def kernel(seq_len, table):
    raise NotImplementedError("write your pallas kernel here")


# speedup vs baseline: 1.7074x; 1.7074x over previous
import functools

import jax
import jax.numpy as jnp
from jax import lax
from jax.experimental import pallas as pl
from jax.experimental.pallas import tpu as pltpu
from jax.experimental.pallas import tpu_sc as plsc

_NUM_CORES = 2
_NUM_SUBCORES = 16
_NSHIFT = 16
_PLANE = 4096


@functools.partial(jax.jit, static_argnums=(1, 2))
def _expand_bias(rev16, H, S):
    ROWS = H * S
    NW = _NUM_CORES * _NUM_SUBCORES
    RPW = ROWS // NW
    WINDOW = 64

    mesh = plsc.VectorSubcoreMesh(core_axis_name="c", subcore_axis_name="s")

    @functools.partial(
        pl.kernel,
        out_type=jax.ShapeDtypeStruct((ROWS * S,), jnp.float32),
        mesh=mesh,
        scratch_types=[pltpu.SemaphoreType.DMA],
        compiler_params=pltpu.CompilerParams(use_tc_tiling_on_sc=False),
    )
    def body(rev_hbm, out_hbm, sem):
        wid = lax.axis_index("s") * _NUM_CORES + lax.axis_index("c")
        base = wid * RPW

        def issue(t, carry):
            r = base + t
            h = r // S
            i = r - h * S
            start = (S - 1) - i
            s = start % _NSHIFT
            q = start // _NSHIFT
            src_off = (s * H + h) * _PLANE + q * _NSHIFT
            pltpu.make_async_copy(
                rev_hbm.at[pl.ds(src_off, S)],
                out_hbm.at[pl.ds(r * S, S)],
                sem,
            ).start()

            @pl.when(t >= WINDOW)
            def _wait_one():
                pltpu.make_async_copy(
                    out_hbm.at[pl.ds(base * S, S)],
                    out_hbm.at[pl.ds(base * S, S)],
                    sem,
                ).wait()

            return carry

        lax.fori_loop(0, RPW, issue, 0)
        pltpu.make_async_copy(
            out_hbm.at[pl.ds(base * S, WINDOW * S)],
            out_hbm.at[pl.ds(base * S, WINDOW * S)],
            sem,
        ).wait()

    return body(rev16)


def kernel(seq_len, table):
    del seq_len
    R, H = table.shape
    S = (R + 1) // 2
    rev = table[::-1, :].T
    rev_pad = jnp.pad(rev, ((0, 0), (0, _PLANE + _NSHIFT - 1 - rev.shape[1])))
    rev16 = jnp.stack([rev_pad[:, s:s + _PLANE] for s in range(_NSHIFT)])
    rows = _expand_bias(rev16.reshape(-1), H, S)
    return rows.reshape(1, H, S, S)


# speedup vs baseline: 1.7097x; 1.0014x over previous
import functools

import jax
import jax.numpy as jnp
from jax import lax
from jax.experimental import pallas as pl
from jax.experimental.pallas import tpu as pltpu
from jax.experimental.pallas import tpu_sc as plsc

_NUM_CORES = 2
_NUM_SUBCORES = 16
_NSHIFT = 16
_PLANE = 4096


@functools.partial(jax.jit, static_argnums=(1, 2))
def _expand_bias(rev16, H, S):
    NW = _NUM_CORES * _NUM_SUBCORES
    IPW = S // NW
    WINDOW = 16

    mesh = plsc.VectorSubcoreMesh(core_axis_name="c", subcore_axis_name="s")

    @functools.partial(
        pl.kernel,
        out_type=jax.ShapeDtypeStruct((H, S, S), jnp.float32),
        mesh=mesh,
        scratch_types=[pltpu.SemaphoreType.DMA],
        compiler_params=pltpu.CompilerParams(use_tc_tiling_on_sc=False),
    )
    def body(rev_hbm, out_hbm, sem):
        wid = lax.axis_index("s") * _NUM_CORES + lax.axis_index("c")
        base = wid * IPW

        def issue(t, carry):
            i = base + t
            start = (S - 1) - i
            s = start % _NSHIFT
            q = start // _NSHIFT
            pltpu.make_async_copy(
                rev_hbm.at[s, :, pl.ds(q * _NSHIFT, S)],
                out_hbm.at[:, i, :],
                sem,
            ).start()

            @pl.when(t >= WINDOW)
            def _wait_one():
                pltpu.make_async_copy(
                    out_hbm.at[:, base, :],
                    out_hbm.at[:, base, :],
                    sem,
                ).wait()

            return carry

        lax.fori_loop(0, IPW, issue, 0)
        pltpu.make_async_copy(
            out_hbm.at[:, pl.ds(base, WINDOW), :],
            out_hbm.at[:, pl.ds(base, WINDOW), :],
            sem,
        ).wait()

    return body(rev16)


def kernel(seq_len, table):
    del seq_len
    R, H = table.shape
    S = (R + 1) // 2
    rev = table[::-1, :].T
    rev_pad = jnp.pad(rev, ((0, 0), (0, _PLANE + _NSHIFT - 1 - rev.shape[1])))
    rev16 = jnp.stack([rev_pad[:, s:s + _PLANE] for s in range(_NSHIFT)])
    rows = _expand_bias(rev16, H, S)
    return rows.reshape(1, H, S, S)


# speedup vs baseline: 31.9459x; 18.6850x over previous
import functools

import jax
import jax.numpy as jnp
from jax import lax
from jax.experimental import pallas as pl
from jax.experimental.pallas import tpu as pltpu
from jax.experimental.pallas import tpu_sc as plsc

_NUM_CORES = 2
_NUM_SUBCORES = 16
_NSHIFT = 16
_PLANE = 4096
_K = 8


@functools.partial(jax.jit, static_argnums=(1, 2))
def _expand_bias(rev16, H, S):
    ROWS = H * S
    NW = _NUM_CORES * _NUM_SUBCORES
    RPW = ROWS // NW
    C = RPW // _K
    C2 = C // 2

    mesh = plsc.VectorSubcoreMesh(core_axis_name="c", subcore_axis_name="s")

    @functools.partial(
        pl.kernel,
        out_type=jax.ShapeDtypeStruct((H, S, S), jnp.float32),
        mesh=mesh,
        scratch_types=[
            pltpu.VMEM((2 * _K, S), jnp.float32),
            pltpu.SemaphoreType.DMA,
            pltpu.SemaphoreType.DMA,
        ],
        compiler_params=pltpu.CompilerParams(use_tc_tiling_on_sc=False),
    )
    def body(rev_hbm, out_hbm, buf, gsem, ssem):
        wid = lax.axis_index("s") * _NUM_CORES + lax.axis_index("c")
        h = wid // 2
        i0 = (wid % 2) * RPW

        def start_gather(t, slot):
            i = i0 + t
            start = (S - 1) - i
            s = start % _NSHIFT
            q = start // _NSHIFT
            pltpu.make_async_copy(
                rev_hbm.at[s, h, pl.ds(q * _NSHIFT, S)], buf.at[slot], gsem
            ).start()

        def start_scatter(t, slot):
            i = i0 + t
            pltpu.make_async_copy(buf.at[slot], out_hbm.at[h, i], ssem).start()

        def wait_gather_one():
            pltpu.make_async_copy(
                rev_hbm.at[0, 0, pl.ds(0, S)], buf.at[0], gsem
            ).wait()

        def wait_scatter_one():
            pltpu.make_async_copy(
                out_hbm.at[0, 0], out_hbm.at[0, 0], ssem
            ).wait()

        def issue_gathers(c, setoff):
            for b in range(_K):
                start_gather(c * _K + b, setoff + b)

        def process_chunk(c, setoff):
            for b in range(_K):
                wait_gather_one()
                start_scatter(c * _K + b, setoff + b)

        def drain_scatters():
            for b in range(_K):
                wait_scatter_one()

        issue_gathers(0, 0)

        def pair(c2, carry):
            c = 2 * c2
            @pl.when(c2 > 0)
            def _d0():
                drain_scatters()
            issue_gathers(c + 1, _K)
            process_chunk(c, 0)
            @pl.when(c2 + 1 < C2)
            def _d1():
                drain_scatters()
                issue_gathers(c + 2, 0)
            process_chunk(c + 1, _K)
            return carry

        lax.fori_loop(0, C2, pair, 0)
        drain_scatters()
        drain_scatters()

    return body(rev16)


def kernel(seq_len, table):
    del seq_len
    R, H = table.shape
    S = (R + 1) // 2
    rev = table[::-1, :].T
    rev_pad = jnp.pad(rev, ((0, 0), (0, _PLANE + _NSHIFT - 1 - rev.shape[1])))
    rev16 = jnp.stack([rev_pad[:, s:s + _PLANE] for s in range(_NSHIFT)])
    rows = _expand_bias(rev16, H, S)
    return rows.reshape(1, H, S, S)


# speedup vs baseline: 41.7016x; 1.3054x over previous
import functools

import jax
import jax.numpy as jnp
from jax import lax
from jax.experimental import pallas as pl
from jax.experimental.pallas import tpu as pltpu
from jax.experimental.pallas import tpu_sc as plsc

_NUM_CORES = 2
_NUM_SUBCORES = 16
_NSHIFT = 16
_PLANE = 4096
_BPW = 64
_WIN = 16 * (_BPW - 1) + 2048


@functools.partial(jax.jit, static_argnums=(1, 2))
def _expand_bias(rev16, H, S):
    mesh = plsc.VectorSubcoreMesh(core_axis_name="c", subcore_axis_name="s")

    @functools.partial(
        pl.kernel,
        out_type=jax.ShapeDtypeStruct((H, S, S), jnp.float32),
        mesh=mesh,
        scratch_types=[
            pltpu.VMEM((H, _WIN), jnp.float32),
            pltpu.SemaphoreType.DMA,
            pltpu.SemaphoreType.DMA,
        ],
        compiler_params=pltpu.CompilerParams(use_tc_tiling_on_sc=False),
    )
    def body(rev_hbm, out_hbm, buf, gsem, ssem):
        wid = lax.axis_index("s") * _NUM_CORES + lax.axis_index("c")
        rho = wid % _NSHIFT
        half = wid // _NSHIFT
        b0 = half * _BPW
        s = (_NSHIFT - 1) - rho
        qmin = (S // _NSHIFT) - b0 - _BPW

        pltpu.make_async_copy(
            rev_hbm.at[s, :, pl.ds(qmin * _NSHIFT, _WIN)], buf, gsem
        ).start()
        pltpu.make_async_copy(
            rev_hbm.at[0, :, pl.ds(0, _WIN)], buf, gsem
        ).wait()

        def issue(t, carry):
            i = rho + _NSHIFT * (b0 + t)
            pltpu.make_async_copy(
                buf.at[:, pl.ds(_NSHIFT * (_BPW - 1 - t), S)],
                out_hbm.at[:, i, :],
                ssem,
            ).start()
            return carry

        lax.fori_loop(0, _BPW, issue, 0)
        pltpu.make_async_copy(
            out_hbm.at[:, pl.ds(0, _BPW), :],
            out_hbm.at[:, pl.ds(0, _BPW), :],
            ssem,
        ).wait()

    return body(rev16)


def kernel(seq_len, table):
    del seq_len
    R, H = table.shape
    S = (R + 1) // 2
    rev = table[::-1, :].T
    rev_pad = jnp.pad(rev, ((0, 0), (0, _PLANE + _NSHIFT - 1 - rev.shape[1])))
    rev16 = jnp.stack([rev_pad[:, s:s + _PLANE] for s in range(_NSHIFT)])
    rows = _expand_bias(rev16, H, S)
    return rows.reshape(1, H, S, S)
